# baseline (speedup 1.0000x reference)
import jax
from jax.experimental import pallas as pl
from jax.experimental.pallas import tpu as pltpu

_ROWS = 4 * 1024
_COLS = 768
_NCHUNK = 16
_CROWS = _ROWS // _NCHUNK


def _identity_copy(x_ref, o_ref, buf, in_sems, out_sems):
    for i in range(_NCHUNK):
        sl = pl.ds(i * _CROWS, _CROWS)
        pltpu.make_async_copy(x_ref.at[sl], buf.at[sl], in_sems.at[i]).start()
    for i in range(_NCHUNK):
        sl = pl.ds(i * _CROWS, _CROWS)
        pltpu.make_async_copy(x_ref.at[sl], buf.at[sl], in_sems.at[i]).wait()
        pltpu.make_async_copy(buf.at[sl], o_ref.at[sl], out_sems.at[i]).start()
    for i in range(_NCHUNK):
        sl = pl.ds(i * _CROWS, _CROWS)
        pltpu.make_async_copy(buf.at[sl], o_ref.at[sl], out_sems.at[i]).wait()


def kernel(x, H, W):
    x2 = x.reshape(_ROWS, _COLS)
    y = pl.pallas_call(
        _identity_copy,
        out_shape=jax.ShapeDtypeStruct((_ROWS, _COLS), x.dtype),
        in_specs=[pl.BlockSpec(memory_space=pl.ANY)],
        out_specs=pl.BlockSpec(memory_space=pl.ANY),
        scratch_shapes=[
            pltpu.VMEM((_ROWS, _COLS), x.dtype),
            pltpu.SemaphoreType.DMA((_NCHUNK,)),
            pltpu.SemaphoreType.DMA((_NCHUNK,)),
        ],
    )(x2)
    return (y.reshape(x.shape), H, W)


# speedup vs baseline: 1.0223x; 1.0223x over previous
import jax
from jax.experimental import pallas as pl
from jax.experimental.pallas import tpu as pltpu

_ROWS = 4 * 1024
_COLS = 768
_NCHUNK = 4
_CROWS = _ROWS // _NCHUNK


def _identity_copy(x_ref, o_ref, buf, in_sems, out_sems):
    for i in range(_NCHUNK):
        sl = pl.ds(i * _CROWS, _CROWS)
        pltpu.make_async_copy(x_ref.at[sl], buf.at[sl], in_sems.at[i]).start()
    for i in range(_NCHUNK):
        sl = pl.ds(i * _CROWS, _CROWS)
        pltpu.make_async_copy(x_ref.at[sl], buf.at[sl], in_sems.at[i]).wait()
        pltpu.make_async_copy(buf.at[sl], o_ref.at[sl], out_sems.at[i]).start()
    for i in range(_NCHUNK):
        sl = pl.ds(i * _CROWS, _CROWS)
        pltpu.make_async_copy(buf.at[sl], o_ref.at[sl], out_sems.at[i]).wait()


def kernel(x, H, W):
    x2 = x.reshape(_ROWS, _COLS)
    y = pl.pallas_call(
        _identity_copy,
        out_shape=jax.ShapeDtypeStruct((_ROWS, _COLS), x.dtype),
        in_specs=[pl.BlockSpec(memory_space=pl.ANY)],
        out_specs=pl.BlockSpec(memory_space=pl.ANY),
        scratch_shapes=[
            pltpu.VMEM((_ROWS, _COLS), x.dtype),
            pltpu.SemaphoreType.DMA((_NCHUNK,)),
            pltpu.SemaphoreType.DMA((_NCHUNK,)),
        ],
    )(x2)
    return (y.reshape(x.shape), H, W)


# speedup vs baseline: 1.0248x; 1.0025x over previous
import jax
from jax.experimental import pallas as pl
from jax.experimental.pallas import tpu as pltpu

_ROWS = 4 * 1024
_COLS = 768
_NCHUNK = 8
_CROWS = _ROWS // _NCHUNK


def _identity_copy(x_ref, o_ref, buf, in_sems, out_sems):
    for i in range(_NCHUNK):
        sl = pl.ds(i * _CROWS, _CROWS)
        pltpu.make_async_copy(x_ref.at[sl], buf.at[sl], in_sems.at[i]).start()
    for i in range(_NCHUNK):
        sl = pl.ds(i * _CROWS, _CROWS)
        pltpu.make_async_copy(x_ref.at[sl], buf.at[sl], in_sems.at[i]).wait()
        pltpu.make_async_copy(buf.at[sl], o_ref.at[sl], out_sems.at[i]).start()
    for i in range(_NCHUNK):
        sl = pl.ds(i * _CROWS, _CROWS)
        pltpu.make_async_copy(buf.at[sl], o_ref.at[sl], out_sems.at[i]).wait()


def kernel(x, H, W):
    x2 = x.reshape(_ROWS, _COLS)
    y = pl.pallas_call(
        _identity_copy,
        out_shape=jax.ShapeDtypeStruct((_ROWS, _COLS), x.dtype),
        in_specs=[pl.BlockSpec(memory_space=pl.ANY)],
        out_specs=pl.BlockSpec(memory_space=pl.ANY),
        scratch_shapes=[
            pltpu.VMEM((_ROWS, _COLS), x.dtype),
            pltpu.SemaphoreType.DMA((_NCHUNK,)),
            pltpu.SemaphoreType.DMA((_NCHUNK,)),
        ],
    )(x2)
    return (y.reshape(x.shape), H, W)


# speedup vs baseline: 1.0262x; 1.0014x over previous
import jax
from jax.experimental import pallas as pl
from jax.experimental.pallas import tpu as pltpu

_ROWS = 4 * 1024
_COLS = 768
_NIN = 16
_NOUT = 8
_IROWS = _ROWS // _NIN
_OROWS = _ROWS // _NOUT
_IN_PER_OUT = _NIN // _NOUT


def _identity_copy(x_ref, o_ref, buf, in_sems, out_sems):
    for i in range(_NIN):
        sl = pl.ds(i * _IROWS, _IROWS)
        pltpu.make_async_copy(x_ref.at[sl], buf.at[sl], in_sems.at[i]).start()
    for j in range(_NOUT):
        for k in range(_IN_PER_OUT):
            i = j * _IN_PER_OUT + k
            sl = pl.ds(i * _IROWS, _IROWS)
            pltpu.make_async_copy(x_ref.at[sl], buf.at[sl], in_sems.at[i]).wait()
        osl = pl.ds(j * _OROWS, _OROWS)
        pltpu.make_async_copy(buf.at[osl], o_ref.at[osl], out_sems.at[j]).start()
    for j in range(_NOUT):
        osl = pl.ds(j * _OROWS, _OROWS)
        pltpu.make_async_copy(buf.at[osl], o_ref.at[osl], out_sems.at[j]).wait()


def kernel(x, H, W):
    x2 = x.reshape(_ROWS, _COLS)
    y = pl.pallas_call(
        _identity_copy,
        out_shape=jax.ShapeDtypeStruct((_ROWS, _COLS), x.dtype),
        in_specs=[pl.BlockSpec(memory_space=pl.ANY)],
        out_specs=pl.BlockSpec(memory_space=pl.ANY),
        scratch_shapes=[
            pltpu.VMEM((_ROWS, _COLS), x.dtype),
            pltpu.SemaphoreType.DMA((_NIN,)),
            pltpu.SemaphoreType.DMA((_NOUT,)),
        ],
    )(x2)
    return (y.reshape(x.shape), H, W)
